# baseline (speedup 1.0000x reference)
import functools

import jax
import jax.numpy as jnp
from jax import lax
from jax.experimental import pallas as pl
from jax.experimental.pallas import tpu as pltpu
from jax.experimental.pallas import tpu_sc as plsc

NC = 2
NS = 16
NW = NC * NS

BATCH = 16384
EMB = 64
B_PER_W = BATCH // NW


def _sc_gather(user_table, item_table, user_ids, item_ids):
    mesh = plsc.VectorSubcoreMesh(core_axis_name="c", subcore_axis_name="s")

    @functools.partial(
        pl.kernel,
        mesh=mesh,
        compiler_params=pltpu.CompilerParams(use_tc_tiling_on_sc=False),
        out_type=[
            jax.ShapeDtypeStruct((BATCH, EMB), jnp.float32),
            jax.ShapeDtypeStruct((BATCH, EMB), jnp.float32),
        ],
        scratch_types=[
            pltpu.VMEM((B_PER_W,), jnp.int32),
            pltpu.VMEM((B_PER_W,), jnp.int32),
            pltpu.VMEM((B_PER_W, EMB), jnp.float32),
            pltpu.VMEM((B_PER_W, EMB), jnp.float32),
            pltpu.SemaphoreType.DMA,
            pltpu.SemaphoreType.DMA,
        ],
    )
    def gather_kernel(utab_hbm, itab_hbm, uid_hbm, iid_hbm,
                      uemb_hbm, iemb_hbm,
                      uidx_v, iidx_v, urows_v, irows_v, usem, isem):
        wid = lax.axis_index("s") * NC + lax.axis_index("c")
        base = wid * B_PER_W
        pltpu.sync_copy(uid_hbm.at[pl.ds(base, B_PER_W)], uidx_v)
        pltpu.sync_copy(iid_hbm.at[pl.ds(base, B_PER_W)], iidx_v)
        cp_u = pltpu.async_copy(utab_hbm.at[uidx_v], urows_v, usem)
        cp_i = pltpu.async_copy(itab_hbm.at[iidx_v], irows_v, isem)
        cp_u.wait()
        pltpu.sync_copy(urows_v, uemb_hbm.at[pl.ds(base, B_PER_W)])
        cp_i.wait()
        pltpu.sync_copy(irows_v, iemb_hbm.at[pl.ds(base, B_PER_W)])

    return gather_kernel(user_table, item_table, user_ids, item_ids)


BM = 1024


def _mlp_body(ue_ref, ie_ref, w1_ref, b1_ref, w2_ref, b2_ref, w3_ref, b3_ref,
              o_ref):
    ue = ue_ref[...]
    ie = ie_ref[...]
    h = jnp.dot(ue, w1_ref[0:EMB, :], preferred_element_type=jnp.float32)
    h = h + jnp.dot(ie, w1_ref[EMB:2 * EMB, :],
                    preferred_element_type=jnp.float32)
    h = jnp.maximum(h + b1_ref[...], 0.0)
    h2 = jnp.dot(h, w2_ref[...], preferred_element_type=jnp.float32)
    h2 = jnp.maximum(h2 + b2_ref[...], 0.0)
    out = jnp.dot(h2, w3_ref[...], preferred_element_type=jnp.float32)
    o_ref[...] = out + b3_ref[...]


def _tc_mlp(uemb, iemb, W1, b1, W2, b2, W3, b3):
    h1 = W1.shape[1]
    h2 = W2.shape[1]
    grid = (BATCH // BM,)
    out = pl.pallas_call(
        _mlp_body,
        grid=grid,
        in_specs=[
            pl.BlockSpec((BM, EMB), lambda i: (i, 0)),
            pl.BlockSpec((BM, EMB), lambda i: (i, 0)),
            pl.BlockSpec((2 * EMB, h1), lambda i: (0, 0)),
            pl.BlockSpec((1, h1), lambda i: (0, 0)),
            pl.BlockSpec((h1, h2), lambda i: (0, 0)),
            pl.BlockSpec((1, h2), lambda i: (0, 0)),
            pl.BlockSpec((h2, 1), lambda i: (0, 0)),
            pl.BlockSpec((1, 1), lambda i: (0, 0)),
        ],
        out_specs=pl.BlockSpec((BM, 1), lambda i: (i, 0)),
        out_shape=jax.ShapeDtypeStruct((BATCH, 1), jnp.float32),
    )(uemb, iemb, W1, b1.reshape(1, h1), W2, b2.reshape(1, h2), W3,
      b3.reshape(1, 1))
    return out.reshape(BATCH)


def kernel(user_ids, item_ids, user_table, item_table, W1, b1, W2, b2, W3, b3):
    uemb, iemb = _sc_gather(user_table, item_table,
                            user_ids.astype(jnp.int32),
                            item_ids.astype(jnp.int32))
    return _tc_mlp(uemb, iemb, W1, b1, W2, b2, W3, b3)
